# baseline (speedup 1.0000x reference)
import functools

import jax
import jax.numpy as jnp
from jax import lax
from jax.experimental import pallas as pl
from jax.experimental.pallas import tpu as pltpu
from jax.experimental.pallas import tpu_sc as plsc

NC = 2
NS = 16
NW = NC * NS
L = 16
C = 128


def _rel_transform_body(R, nf_ref, w_ref, out_ref):
    nf = nf_ref[...]
    for r in range(R):
        out_ref[r] = jnp.dot(nf, w_ref[r], preferred_element_type=jnp.float32)


def _rel_transform(node_feat, W_rel):
    R, D, _ = W_rel.shape
    N = node_feat.shape[0]
    BN = 1000
    return pl.pallas_call(
        functools.partial(_rel_transform_body, R),
        grid=(N // BN,),
        in_specs=[
            pl.BlockSpec((BN, D), lambda i: (i, 0)),
            pl.BlockSpec((R, D, D), lambda i: (0, 0, 0)),
        ],
        out_specs=pl.BlockSpec((R, BN, D), lambda i: (0, i, 0)),
        out_shape=jax.ShapeDtypeStruct((R, N, D), jnp.float32),
    )(node_feat, W_rel)


def _sc_body(NA, NB, AGG_ROWS, Q, QPT, QPW, D, CAP,
             xrel_hbm, idx_hbm, qidx_hbm, nf_hbm,
             ga_hbm, gnf_hbm,
             agg_sh, glist, dlist, qstage, idx0, idx1, rows0, rows1,
             wmap, dvec, gvec0, gvec1, qv, qv2,
             semi0, semi1, semg0, semg1, sem):
    cid = lax.axis_index("c")
    sid = lax.axis_index("s")
    wid = sid * NC + cid
    idxb = (idx0, idx1)
    rowsb = (rows0, rows1)
    semib = (semi0, semi1)
    semgb = (semg0, semg1)
    gvecs = (gvec0, gvec1)
    G = rows0.shape[0]
    z = jnp.zeros((L,), jnp.float32)
    nseg = D // L
    padrow = jnp.full((L,), AGG_ROWS - 1, jnp.int32)

    def zstore(i, _):
        rows0[i // nseg, pl.ds((i % nseg) * L, L)] = z
        return 0

    lax.fori_loop(0, G * nseg, zstore, 0)

    rows_per_tile = AGG_ROWS // NS
    tb = sid * rows_per_tile
    nz = rows_per_tile // G

    def zfire(j, _):
        pltpu.async_copy(rows0, agg_sh.at[pl.ds(tb + j * G, G)], sem)
        return 0

    lax.fori_loop(0, nz, zfire, 0)

    for w in range(wmap.shape[0] // L):
        wmap[pl.ds(w * L, L)] = jnp.zeros((L,), jnp.int32)

    pltpu.sync_copy(qidx_hbm, qstage)
    lane0 = lax.iota(jnp.int32, L) == 0
    neg = jnp.full((L,), -2147483648, jnp.int32)

    def bset(i, _):
        base2 = jnp.minimum(i, Q - L)
        v = qstage[pl.ds(base2, L)]
        laneq = lax.iota(jnp.int32, L) == (i - base2)
        q = jnp.max(jnp.where(laneq, v, neg))
        w = q >> 5
        vw = wmap[pl.ds(w, L)]
        addv = jnp.where(lane0, jnp.int32(1) << (q & 31), jnp.int32(0))
        wmap[pl.ds(w, L)] = vw | addv
        return 0

    lax.fori_loop(0, Q, bset, 0)

    CH = jnp.where(cid == 0, NA, NB)
    base = jnp.where(cid == 0, sid * NA, NS * NA + sid * NB)

    def fire_idx(ch, b):
        pltpu.async_copy(idx_hbm.at[base + ch], idxb[b], semib[b])

    def wait_idx(ch, b):
        pltpu.make_async_copy(idx_hbm.at[base + ch], idxb[b],
                              semib[b]).wait()

    fire_idx(0, 0)
    fire_idx(1, 1)

    def fchunk(ch, b, off):
        wait_idx(ch, b)
        for s in range(C // L):
            vg = idxb[b][0, pl.ds(s * L, L)]
            vd = idxb[b][1, pl.ds(s * L, L)]
            w = plsc.load_gather(wmap, [vd >> 5])
            keep = ((w >> (vd & 31)) & 1) == 1
            plsc.store_compressed(glist.at[pl.ds(off, L)], vg, mask=keep)
            plsc.store_compressed(dlist.at[pl.ds(off, L)], vd, mask=keep)
            off = off + jnp.sum(keep.astype(jnp.int32))

        @pl.when(ch + 2 < CH)
        def _():
            fire_idx(ch + 2, b)

        return off

    def fchunk2(g, off):
        off = fchunk(2 * g, 0, off)
        off = fchunk(2 * g + 1, 1, off)
        return off

    off = lax.fori_loop(0, CH // 2, fchunk2, jnp.int32(0))

    for k in range(2 * G // L):
        glist[pl.ds(off + k * L, L)] = jnp.zeros((L,), jnp.int32)
        dlist[pl.ds(off + k * L, L)] = padrow

    def zdrain(j, _):
        pltpu.make_async_copy(rows0, agg_sh.at[pl.ds(tb + j * G, G)],
                              sem).wait()
        return 0

    lax.fori_loop(0, nz, zdrain, 0)
    plsc.subcore_barrier()

    KC2 = (off + 2 * G - 1) // (2 * G)
    KCT = 2 * KC2

    def prep_gvec(ch, b):
        for s in range(G // L):
            gvecs[b][0, pl.ds(s * L, L)] = glist[pl.ds(ch * G + s * L, L)]

    def fire_gather(ch, b):
        prep_gvec(ch, b)
        pltpu.async_copy(xrel_hbm.at[gvecs[b].at[0]], rowsb[b], semgb[b])

    def wait_gather(b):
        pltpu.make_async_copy(xrel_hbm.at[gvecs[b].at[0]], rowsb[b],
                              semgb[b]).wait()

    @pl.when(KCT > 0)
    def _():
        fire_gather(0, 0)

    def gchunk(ch, b):
        nb = 1 - b
        wait_gather(b)

        @pl.when(ch + 1 < KCT)
        def _():
            fire_gather(ch + 1, nb)

        for s in range(G // L):
            dvec[0, pl.ds(s * L, L)] = dlist[pl.ds(ch * G + s * L, L)]
        pltpu.sync_copy(rowsb[b], agg_sh.at[dvec.at[0]], add=True)

    def gchunk2(g, _):
        gchunk(2 * g, 0)
        gchunk(2 * g + 1, 1)
        return 0

    lax.fori_loop(0, KC2, gchunk2, 0)
    plsc.subcore_barrier()

    qb = sid * QPT
    pltpu.sync_copy(qidx_hbm.at[pl.ds(qb, QPT)], qv)
    for h in range(QPT // G):
        pltpu.async_copy(agg_sh.at[qv.at[pl.ds(h * G, G)]], rowsb[h % 2],
                         semgb[h % 2])
    for h in range(QPT // G):
        pltpu.make_async_copy(agg_sh.at[qv.at[pl.ds(h * G, G)]],
                              rowsb[h % 2], semgb[h % 2]).wait()
        pltpu.sync_copy(rowsb[h % 2], ga_hbm.at[cid, pl.ds(qb + h * G, G)])

    qb2 = wid * QPW
    pltpu.sync_copy(qidx_hbm.at[pl.ds(qb2, QPW)], qv2)
    pltpu.async_copy(nf_hbm.at[qv2], rows0.at[pl.ds(0, QPW)], sem).wait()
    pltpu.sync_copy(rows0.at[pl.ds(0, QPW)], gnf_hbm.at[pl.ds(qb2, QPW)])


def _sc_aggregate(x_rel, idx, qidx, node_feat, AGG_ROWS, NA, NB):
    D = node_feat.shape[1]
    Q = qidx.shape[0]
    QPT = Q // NS
    QPW = Q // NW
    G = 64
    CAP = max(NA, NB) * C + 2 * G
    mesh = plsc.VectorSubcoreMesh(core_axis_name="c", subcore_axis_name="s",
                                  num_cores=NC, num_subcores=NS)
    body = functools.partial(_sc_body, NA, NB, AGG_ROWS, Q, QPT, QPW, D, CAP)
    f = pl.kernel(
        body,
        out_type=[
            jax.ShapeDtypeStruct((NC, Q, D), jnp.float32),
            jax.ShapeDtypeStruct((Q, D), jnp.float32),
        ],
        mesh=mesh,
        compiler_params=pltpu.CompilerParams(needs_layout_passes=False),
        scratch_types=[
            pltpu.VMEM_SHARED((AGG_ROWS, D), jnp.float32),
            pltpu.VMEM((CAP,), jnp.int32),
            pltpu.VMEM((CAP,), jnp.int32),
            pltpu.VMEM((Q,), jnp.int32),
            pltpu.VMEM((2, C), jnp.int32),
            pltpu.VMEM((2, C), jnp.int32),
            pltpu.VMEM((G, D), jnp.float32),
            pltpu.VMEM((G, D), jnp.float32),
            pltpu.VMEM((AGG_ROWS // 32 + L,), jnp.int32),
            pltpu.VMEM((1, G), jnp.int32),
            pltpu.VMEM((1, G), jnp.int32),
            pltpu.VMEM((1, G), jnp.int32),
            pltpu.VMEM((QPT,), jnp.int32),
            pltpu.VMEM((QPW,), jnp.int32),
            pltpu.SemaphoreType.DMA,
            pltpu.SemaphoreType.DMA,
            pltpu.SemaphoreType.DMA,
            pltpu.SemaphoreType.DMA,
            pltpu.SemaphoreType.DMA,
        ],
    )
    return f(x_rel, idx, qidx, node_feat)


def _head_body(ga_ref, gnf_ref, wr_ref, bg_ref, wf_ref, bf_ref, out_ref):
    D = wr_ref.shape[0]
    Bq = out_ref.shape[0]
    t = (ga_ref[0] + ga_ref[1]
         + jnp.dot(gnf_ref[...], wr_ref[...],
                   preferred_element_type=jnp.float32)
         + bg_ref[...])
    t = jnp.maximum(t, 0.0)
    hid = (jnp.dot(t[:Bq], wf_ref[:D], preferred_element_type=jnp.float32)
           + jnp.dot(t[Bq:], wf_ref[D:], preferred_element_type=jnp.float32)
           + bf_ref[...])
    out_ref[...] = jnp.maximum(hid, 0.0)


def _head(ga, gnf, W_root, b_gnn, W_fc, b_fc):
    B2 = ga.shape[1]
    H = W_fc.shape[1]
    return pl.pallas_call(
        _head_body,
        out_shape=jax.ShapeDtypeStruct((B2 // 2, H), jnp.float32),
    )(ga, gnf, W_root, b_gnn.reshape(1, -1), W_fc, b_fc.reshape(1, -1))


def kernel(x, node_feat, edge_index, edge_type, nest_tensor, food_tensor,
           W_rel, W_root, b_gnn, W_fc, b_fc):
    N, D = node_feat.shape
    R = W_rel.shape[0]
    E = edge_type.shape[0]

    src = edge_index[0].astype(jnp.int32)
    dst = edge_index[1].astype(jnp.int32)
    et = edge_type.astype(jnp.int32)

    x_rel = _rel_transform(node_feat, W_rel).reshape(R * N, D)

    T16 = -(-E // (NS * C))
    NA = max(2, 2 * round(T16 * 0.65 / 2))
    NB = max(2, 2 * (-(-(T16 - NA) // 2)))
    T = NS * (NA + NB)
    pad = T * C - E
    AGG_ROWS = -(-N // (64 * NS)) * (64 * NS)

    gidx = jnp.concatenate([et * N + src,
                            jnp.zeros((pad,), jnp.int32)]).reshape(T, C)
    didx = jnp.concatenate([dst,
                            jnp.full((pad,), AGG_ROWS - 1, jnp.int32)
                            ]).reshape(T, C)
    idx = jnp.stack([gidx, didx], axis=1)
    qidx = jnp.concatenate([nest_tensor.astype(jnp.int32),
                            food_tensor.astype(jnp.int32)])

    ga, gnf = _sc_aggregate(x_rel, idx, qidx, node_feat, AGG_ROWS, NA, NB)

    return _head(ga, gnf, W_root, b_gnn, W_fc, b_fc)


# speedup vs baseline: 1.0177x; 1.0177x over previous
import functools

import jax
import jax.numpy as jnp
from jax import lax
from jax.experimental import pallas as pl
from jax.experimental.pallas import tpu as pltpu
from jax.experimental.pallas import tpu_sc as plsc

NC = 2
NS = 16
NW = NC * NS
L = 16
C = 128


def _rel_transform_body(R, nf_ref, w_ref, out_ref):
    nf = nf_ref[...]
    for r in range(R):
        out_ref[r] = jnp.dot(nf, w_ref[r], preferred_element_type=jnp.float32)


def _rel_transform(node_feat, W_rel):
    R, D, _ = W_rel.shape
    N = node_feat.shape[0]
    BN = 1000
    return pl.pallas_call(
        functools.partial(_rel_transform_body, R),
        grid=(N // BN,),
        in_specs=[
            pl.BlockSpec((BN, D), lambda i: (i, 0)),
            pl.BlockSpec((R, D, D), lambda i: (0, 0, 0)),
        ],
        out_specs=pl.BlockSpec((R, BN, D), lambda i: (0, i, 0)),
        out_shape=jax.ShapeDtypeStruct((R, N, D), jnp.float32),
    )(node_feat, W_rel)


def _sc_body(NA, NB, AGG_ROWS, Q, QPT, QPW, D, CAP,
             xrel_hbm, idx_hbm, qidx_hbm, nf_hbm,
             ga_hbm, gnf_hbm,
             agg_sh, glist, dlist, qstage, idx0, idx1, rows0, rows1,
             wmap, dvec, gvec0, gvec1, qv, qv2,
             semi0, semi1, semg0, semg1, sem):
    cid = lax.axis_index("c")
    sid = lax.axis_index("s")
    wid = sid * NC + cid
    idxb = (idx0, idx1)
    rowsb = (rows0, rows1)
    semib = (semi0, semi1)
    semgb = (semg0, semg1)
    gvecs = (gvec0, gvec1)
    G = rows0.shape[0]
    z = jnp.zeros((L,), jnp.float32)
    nseg = D // L
    padrow = jnp.full((L,), AGG_ROWS - 1, jnp.int32)

    def zstore(i, _):
        rows0[i // nseg, pl.ds((i % nseg) * L, L)] = z
        return 0

    lax.fori_loop(0, G * nseg, zstore, 0)

    rows_per_tile = AGG_ROWS // NS
    tb = sid * rows_per_tile
    nz = rows_per_tile // G

    def zfire(j, _):
        pltpu.async_copy(rows0, agg_sh.at[pl.ds(tb + j * G, G)], sem)
        return 0

    lax.fori_loop(0, nz, zfire, 0)

    for w in range(wmap.shape[0] // L):
        wmap[pl.ds(w * L, L)] = jnp.zeros((L,), jnp.int32)

    pltpu.sync_copy(qidx_hbm, qstage)
    lane0 = lax.iota(jnp.int32, L) == 0
    neg = jnp.full((L,), -2147483648, jnp.int32)

    def bset(i, _):
        base2 = jnp.minimum(i, Q - L)
        v = qstage[pl.ds(base2, L)]
        laneq = lax.iota(jnp.int32, L) == (i - base2)
        q = jnp.max(jnp.where(laneq, v, neg))
        w = q >> 5
        vw = wmap[pl.ds(w, L)]
        addv = jnp.where(lane0, jnp.int32(1) << (q & 31), jnp.int32(0))
        wmap[pl.ds(w, L)] = vw | addv
        return 0

    lax.fori_loop(0, Q, bset, 0)

    CH = jnp.where(cid == 0, NA, NB)
    base = jnp.where(cid == 0, sid * NA, NS * NA + sid * NB)

    def fire_idx(ch, b):
        pltpu.async_copy(idx_hbm.at[base + ch], idxb[b], semib[b])

    def wait_idx(ch, b):
        pltpu.make_async_copy(idx_hbm.at[base + ch], idxb[b],
                              semib[b]).wait()

    fire_idx(0, 0)
    fire_idx(1, 1)

    def fchunk(ch, b, off):
        wait_idx(ch, b)
        vgs, vds, keeps, cnts = [], [], [], []
        for s in range(C // L):
            vg = idxb[b][0, pl.ds(s * L, L)]
            vd = idxb[b][1, pl.ds(s * L, L)]
            w = plsc.load_gather(wmap, [vd >> 5])
            keep = ((w >> (vd & 31)) & 1) == 1
            vgs.append(vg)
            vds.append(vd)
            keeps.append(keep)
            cnts.append(jnp.sum(keep.astype(jnp.int32)))
        offs = [off]
        for s in range(1, C // L):
            offs.append(offs[-1] + cnts[s - 1])
        for s in range(C // L):
            plsc.store_compressed(glist.at[pl.ds(offs[s], L)], vgs[s],
                                  mask=keeps[s])
            plsc.store_compressed(dlist.at[pl.ds(offs[s], L)], vds[s],
                                  mask=keeps[s])

        @pl.when(ch + 2 < CH)
        def _():
            fire_idx(ch + 2, b)

        return offs[-1] + cnts[-1]

    def fchunk2(g, off):
        off = fchunk(2 * g, 0, off)
        off = fchunk(2 * g + 1, 1, off)
        return off

    off = lax.fori_loop(0, CH // 2, fchunk2, jnp.int32(0))

    for k in range(2 * G // L):
        glist[pl.ds(off + k * L, L)] = jnp.zeros((L,), jnp.int32)
        dlist[pl.ds(off + k * L, L)] = padrow

    def zdrain(j, _):
        pltpu.make_async_copy(rows0, agg_sh.at[pl.ds(tb + j * G, G)],
                              sem).wait()
        return 0

    lax.fori_loop(0, nz, zdrain, 0)
    plsc.subcore_barrier()

    KC2 = (off + 2 * G - 1) // (2 * G)
    KCT = 2 * KC2

    def prep_gvec(ch, b):
        for s in range(G // L):
            gvecs[b][0, pl.ds(s * L, L)] = glist[pl.ds(ch * G + s * L, L)]

    def fire_gather(ch, b):
        prep_gvec(ch, b)
        pltpu.async_copy(xrel_hbm.at[gvecs[b].at[0]], rowsb[b], semgb[b])

    def wait_gather(b):
        pltpu.make_async_copy(xrel_hbm.at[gvecs[b].at[0]], rowsb[b],
                              semgb[b]).wait()

    @pl.when(KCT > 0)
    def _():
        fire_gather(0, 0)

    def gchunk(ch, b):
        nb = 1 - b
        wait_gather(b)

        @pl.when(ch + 1 < KCT)
        def _():
            fire_gather(ch + 1, nb)

        for s in range(G // L):
            dvec[0, pl.ds(s * L, L)] = dlist[pl.ds(ch * G + s * L, L)]
        pltpu.sync_copy(rowsb[b], agg_sh.at[dvec.at[0]], add=True)

    def gchunk2(g, _):
        gchunk(2 * g, 0)
        gchunk(2 * g + 1, 1)
        return 0

    lax.fori_loop(0, KC2, gchunk2, 0)
    plsc.subcore_barrier()

    qb = sid * QPT
    pltpu.sync_copy(qidx_hbm.at[pl.ds(qb, QPT)], qv)
    for h in range(QPT // G):
        pltpu.async_copy(agg_sh.at[qv.at[pl.ds(h * G, G)]], rowsb[h % 2],
                         semgb[h % 2])
    for h in range(QPT // G):
        pltpu.make_async_copy(agg_sh.at[qv.at[pl.ds(h * G, G)]],
                              rowsb[h % 2], semgb[h % 2]).wait()
        pltpu.sync_copy(rowsb[h % 2], ga_hbm.at[cid, pl.ds(qb + h * G, G)])

    qb2 = wid * QPW
    pltpu.sync_copy(qidx_hbm.at[pl.ds(qb2, QPW)], qv2)
    pltpu.async_copy(nf_hbm.at[qv2], rows0.at[pl.ds(0, QPW)], sem).wait()
    pltpu.sync_copy(rows0.at[pl.ds(0, QPW)], gnf_hbm.at[pl.ds(qb2, QPW)])


def _sc_aggregate(x_rel, idx, qidx, node_feat, AGG_ROWS, NA, NB):
    D = node_feat.shape[1]
    Q = qidx.shape[0]
    QPT = Q // NS
    QPW = Q // NW
    G = 64
    CAP = max(NA, NB) * C + 2 * G
    mesh = plsc.VectorSubcoreMesh(core_axis_name="c", subcore_axis_name="s",
                                  num_cores=NC, num_subcores=NS)
    body = functools.partial(_sc_body, NA, NB, AGG_ROWS, Q, QPT, QPW, D, CAP)
    f = pl.kernel(
        body,
        out_type=[
            jax.ShapeDtypeStruct((NC, Q, D), jnp.float32),
            jax.ShapeDtypeStruct((Q, D), jnp.float32),
        ],
        mesh=mesh,
        compiler_params=pltpu.CompilerParams(needs_layout_passes=False),
        scratch_types=[
            pltpu.VMEM_SHARED((AGG_ROWS, D), jnp.float32),
            pltpu.VMEM((CAP,), jnp.int32),
            pltpu.VMEM((CAP,), jnp.int32),
            pltpu.VMEM((Q,), jnp.int32),
            pltpu.VMEM((2, C), jnp.int32),
            pltpu.VMEM((2, C), jnp.int32),
            pltpu.VMEM((G, D), jnp.float32),
            pltpu.VMEM((G, D), jnp.float32),
            pltpu.VMEM((AGG_ROWS // 32 + L,), jnp.int32),
            pltpu.VMEM((1, G), jnp.int32),
            pltpu.VMEM((1, G), jnp.int32),
            pltpu.VMEM((1, G), jnp.int32),
            pltpu.VMEM((QPT,), jnp.int32),
            pltpu.VMEM((QPW,), jnp.int32),
            pltpu.SemaphoreType.DMA,
            pltpu.SemaphoreType.DMA,
            pltpu.SemaphoreType.DMA,
            pltpu.SemaphoreType.DMA,
            pltpu.SemaphoreType.DMA,
        ],
    )
    return f(x_rel, idx, qidx, node_feat)


def _head_body(ga_ref, gnf_ref, wr_ref, bg_ref, wf_ref, bf_ref, out_ref):
    D = wr_ref.shape[0]
    Bq = out_ref.shape[0]
    t = (ga_ref[0] + ga_ref[1]
         + jnp.dot(gnf_ref[...], wr_ref[...],
                   preferred_element_type=jnp.float32)
         + bg_ref[...])
    t = jnp.maximum(t, 0.0)
    hid = (jnp.dot(t[:Bq], wf_ref[:D], preferred_element_type=jnp.float32)
           + jnp.dot(t[Bq:], wf_ref[D:], preferred_element_type=jnp.float32)
           + bf_ref[...])
    out_ref[...] = jnp.maximum(hid, 0.0)


def _head(ga, gnf, W_root, b_gnn, W_fc, b_fc):
    B2 = ga.shape[1]
    H = W_fc.shape[1]
    return pl.pallas_call(
        _head_body,
        out_shape=jax.ShapeDtypeStruct((B2 // 2, H), jnp.float32),
    )(ga, gnf, W_root, b_gnn.reshape(1, -1), W_fc, b_fc.reshape(1, -1))


def kernel(x, node_feat, edge_index, edge_type, nest_tensor, food_tensor,
           W_rel, W_root, b_gnn, W_fc, b_fc):
    N, D = node_feat.shape
    R = W_rel.shape[0]
    E = edge_type.shape[0]

    src = edge_index[0].astype(jnp.int32)
    dst = edge_index[1].astype(jnp.int32)
    et = edge_type.astype(jnp.int32)

    x_rel = _rel_transform(node_feat, W_rel).reshape(R * N, D)

    T16 = -(-E // (NS * C))
    NA = max(2, 2 * round(T16 * 0.59 / 2))
    NB = max(2, 2 * (-(-(T16 - NA) // 2)))
    T = NS * (NA + NB)
    pad = T * C - E
    AGG_ROWS = -(-N // (64 * NS)) * (64 * NS)

    gidx = jnp.concatenate([et * N + src,
                            jnp.zeros((pad,), jnp.int32)]).reshape(T, C)
    didx = jnp.concatenate([dst,
                            jnp.full((pad,), AGG_ROWS - 1, jnp.int32)
                            ]).reshape(T, C)
    idx = jnp.stack([gidx, didx], axis=1)
    qidx = jnp.concatenate([nest_tensor.astype(jnp.int32),
                            food_tensor.astype(jnp.int32)])

    ga, gnf = _sc_aggregate(x_rel, idx, qidx, node_feat, AGG_ROWS, NA, NB)

    return _head(ga, gnf, W_root, b_gnn, W_fc, b_fc)


# speedup vs baseline: 1.0496x; 1.0313x over previous
import functools

import jax
import jax.numpy as jnp
from jax import lax
from jax.experimental import pallas as pl
from jax.experimental.pallas import tpu as pltpu
from jax.experimental.pallas import tpu_sc as plsc

NC = 2
NS = 16
NW = NC * NS
L = 16
C = 128


def _rel_transform_body(R, nf_ref, w_ref, out_ref):
    nf = nf_ref[...]
    for r in range(R):
        out_ref[r] = jnp.dot(nf, w_ref[r], preferred_element_type=jnp.float32)


def _rel_transform(node_feat, W_rel):
    R, D, _ = W_rel.shape
    N = node_feat.shape[0]
    BN = 1000
    return pl.pallas_call(
        functools.partial(_rel_transform_body, R),
        grid=(N // BN,),
        in_specs=[
            pl.BlockSpec((BN, D), lambda i: (i, 0)),
            pl.BlockSpec((R, D, D), lambda i: (0, 0, 0)),
        ],
        out_specs=pl.BlockSpec((R, BN, D), lambda i: (0, i, 0)),
        out_shape=jax.ShapeDtypeStruct((R, N, D), jnp.float32),
    )(node_feat, W_rel)


def _sc_body(NA, NB, AGG_ROWS, Q, QPT, QPW, D, CAP,
             xrel_hbm, gidx_hbm, didx_hbm, qidx_hbm, nf_hbm,
             ga_hbm, gnf_hbm,
             agg_sh, glist, dlist, qstage, idxg0, idxg1, idxd0, idxd1,
             rows0, rows1, wmap, dvec, gvec0, gvec1, qv, qv2,
             semi0, semi1, semd0, semd1, semg0, semg1, sem):
    cid = lax.axis_index("c")
    sid = lax.axis_index("s")
    wid = sid * NC + cid
    idxgb = (idxg0, idxg1)
    idxdb = (idxd0, idxd1)
    rowsb = (rows0, rows1)
    semib = (semi0, semi1)
    semdb = (semd0, semd1)
    semgb = (semg0, semg1)
    gvecs = (gvec0, gvec1)
    G = rows0.shape[0]
    z = jnp.zeros((L,), jnp.float32)
    nseg = D // L
    padrow = jnp.full((L,), AGG_ROWS - 1, jnp.int32)

    def zstore(i, _):
        rows0[i // nseg, pl.ds((i % nseg) * L, L)] = z
        return 0

    lax.fori_loop(0, G * nseg, zstore, 0)

    rows_per_tile = AGG_ROWS // NS
    tb = sid * rows_per_tile
    nz = rows_per_tile // G

    def zfire(j, _):
        pltpu.async_copy(rows0, agg_sh.at[pl.ds(tb + j * G, G)], sem)
        return 0

    lax.fori_loop(0, nz, zfire, 0)

    for w in range(wmap.shape[0] // L):
        wmap[pl.ds(w * L, L)] = jnp.zeros((L,), jnp.int32)

    pltpu.sync_copy(qidx_hbm, qstage)
    lane0 = lax.iota(jnp.int32, L) == 0
    neg = jnp.full((L,), -2147483648, jnp.int32)

    def bset(i, _):
        base2 = jnp.minimum(i, Q - L)
        v = qstage[pl.ds(base2, L)]
        laneq = lax.iota(jnp.int32, L) == (i - base2)
        q = jnp.max(jnp.where(laneq, v, neg))
        w = q >> 5
        vw = wmap[pl.ds(w, L)]
        addv = jnp.where(lane0, jnp.int32(1) << (q & 31), jnp.int32(0))
        wmap[pl.ds(w, L)] = vw | addv
        return 0

    lax.fori_loop(0, Q, bset, 0)

    CH = jnp.where(cid == 0, NA, NB)
    base = jnp.where(cid == 0, sid * NA, NS * NA + sid * NB)

    def fire_idx(ch, b):
        pltpu.async_copy(gidx_hbm.at[base + ch], idxgb[b], semib[b])
        pltpu.async_copy(didx_hbm.at[base + ch], idxdb[b], semdb[b])

    def wait_idx(ch, b):
        pltpu.make_async_copy(gidx_hbm.at[base + ch], idxgb[b],
                              semib[b]).wait()
        pltpu.make_async_copy(didx_hbm.at[base + ch], idxdb[b],
                              semdb[b]).wait()

    fire_idx(0, 0)
    fire_idx(1, 1)

    def fchunk(ch, b, off):
        wait_idx(ch, b)
        vgs, vds, keeps, cnts = [], [], [], []
        for s in range(C // L):
            vg = idxgb[b][pl.ds(s * L, L)]
            vd = idxdb[b][pl.ds(s * L, L)]
            w = plsc.load_gather(wmap, [vd >> 5])
            keep = ((w >> (vd & 31)) & 1) == 1
            vgs.append(vg)
            vds.append(vd)
            keeps.append(keep)
            cnts.append(jnp.sum(keep.astype(jnp.int32)))
        offs = [off]
        for s in range(1, C // L):
            offs.append(offs[-1] + cnts[s - 1])
        for s in range(C // L):
            plsc.store_compressed(glist.at[pl.ds(offs[s], L)], vgs[s],
                                  mask=keeps[s])
            plsc.store_compressed(dlist.at[pl.ds(offs[s], L)], vds[s],
                                  mask=keeps[s])

        @pl.when(ch + 2 < CH)
        def _():
            fire_idx(ch + 2, b)

        return offs[-1] + cnts[-1]

    def fchunk2(g, off):
        off = fchunk(2 * g, 0, off)
        off = fchunk(2 * g + 1, 1, off)
        return off

    off = lax.fori_loop(0, CH // 2, fchunk2, jnp.int32(0))

    for k in range(2 * G // L):
        glist[pl.ds(off + k * L, L)] = jnp.zeros((L,), jnp.int32)
        dlist[pl.ds(off + k * L, L)] = padrow

    def zdrain(j, _):
        pltpu.make_async_copy(rows0, agg_sh.at[pl.ds(tb + j * G, G)],
                              sem).wait()
        return 0

    lax.fori_loop(0, nz, zdrain, 0)
    plsc.subcore_barrier()

    KC2 = (off + 2 * G - 1) // (2 * G)
    KCT = 2 * KC2

    def prep_gvec(ch, b):
        for s in range(G // L):
            gvecs[b][0, pl.ds(s * L, L)] = glist[pl.ds(ch * G + s * L, L)]

    def fire_gather(ch, b):
        prep_gvec(ch, b)
        pltpu.async_copy(xrel_hbm.at[gvecs[b].at[0]], rowsb[b], semgb[b])

    def wait_gather(b):
        pltpu.make_async_copy(xrel_hbm.at[gvecs[b].at[0]], rowsb[b],
                              semgb[b]).wait()

    @pl.when(KCT > 0)
    def _():
        fire_gather(0, 0)

    def gchunk(ch, b):
        nb = 1 - b
        wait_gather(b)

        @pl.when(ch + 1 < KCT)
        def _():
            fire_gather(ch + 1, nb)

        for s in range(G // L):
            dvec[0, pl.ds(s * L, L)] = dlist[pl.ds(ch * G + s * L, L)]
        pltpu.sync_copy(rowsb[b], agg_sh.at[dvec.at[0]], add=True)

    def gchunk2(g, _):
        gchunk(2 * g, 0)
        gchunk(2 * g + 1, 1)
        return 0

    lax.fori_loop(0, KC2, gchunk2, 0)
    plsc.subcore_barrier()

    qb = sid * QPT
    pltpu.sync_copy(qidx_hbm.at[pl.ds(qb, QPT)], qv)
    for h in range(QPT // G):
        pltpu.async_copy(agg_sh.at[qv.at[pl.ds(h * G, G)]], rowsb[h % 2],
                         semgb[h % 2])
    for h in range(QPT // G):
        pltpu.make_async_copy(agg_sh.at[qv.at[pl.ds(h * G, G)]],
                              rowsb[h % 2], semgb[h % 2]).wait()
        pltpu.sync_copy(rowsb[h % 2], ga_hbm.at[cid, pl.ds(qb + h * G, G)])

    qb2 = wid * QPW
    pltpu.sync_copy(qidx_hbm.at[pl.ds(qb2, QPW)], qv2)
    pltpu.async_copy(nf_hbm.at[qv2], rows0.at[pl.ds(0, QPW)], sem).wait()
    pltpu.sync_copy(rows0.at[pl.ds(0, QPW)], gnf_hbm.at[pl.ds(qb2, QPW)])


def _sc_aggregate(x_rel, gidx, didx, qidx, node_feat, AGG_ROWS, NA, NB):
    D = node_feat.shape[1]
    Q = qidx.shape[0]
    QPT = Q // NS
    QPW = Q // NW
    G = 64
    CAP = max(NA, NB) * C + 2 * G
    mesh = plsc.VectorSubcoreMesh(core_axis_name="c", subcore_axis_name="s",
                                  num_cores=NC, num_subcores=NS)
    body = functools.partial(_sc_body, NA, NB, AGG_ROWS, Q, QPT, QPW, D, CAP)
    f = pl.kernel(
        body,
        out_type=[
            jax.ShapeDtypeStruct((NC, Q, D), jnp.float32),
            jax.ShapeDtypeStruct((Q, D), jnp.float32),
        ],
        mesh=mesh,
        compiler_params=pltpu.CompilerParams(needs_layout_passes=False),
        scratch_types=[
            pltpu.VMEM_SHARED((AGG_ROWS, D), jnp.float32),
            pltpu.VMEM((CAP,), jnp.int32),
            pltpu.VMEM((CAP,), jnp.int32),
            pltpu.VMEM((Q,), jnp.int32),
            pltpu.VMEM((C,), jnp.int32),
            pltpu.VMEM((C,), jnp.int32),
            pltpu.VMEM((C,), jnp.int32),
            pltpu.VMEM((C,), jnp.int32),
            pltpu.VMEM((G, D), jnp.float32),
            pltpu.VMEM((G, D), jnp.float32),
            pltpu.VMEM((AGG_ROWS // 32 + L,), jnp.int32),
            pltpu.VMEM((1, G), jnp.int32),
            pltpu.VMEM((1, G), jnp.int32),
            pltpu.VMEM((1, G), jnp.int32),
            pltpu.VMEM((QPT,), jnp.int32),
            pltpu.VMEM((QPW,), jnp.int32),
            pltpu.SemaphoreType.DMA,
            pltpu.SemaphoreType.DMA,
            pltpu.SemaphoreType.DMA,
            pltpu.SemaphoreType.DMA,
            pltpu.SemaphoreType.DMA,
            pltpu.SemaphoreType.DMA,
            pltpu.SemaphoreType.DMA,
        ],
    )
    return f(x_rel, gidx, didx, qidx, node_feat)


def _head_body(ga_ref, gnf_ref, wr_ref, bg_ref, wf_ref, bf_ref, out_ref):
    D = wr_ref.shape[0]
    Bq = out_ref.shape[0]
    t = (ga_ref[0] + ga_ref[1]
         + jnp.dot(gnf_ref[...], wr_ref[...],
                   preferred_element_type=jnp.float32)
         + bg_ref[...])
    t = jnp.maximum(t, 0.0)
    hid = (jnp.dot(t[:Bq], wf_ref[:D], preferred_element_type=jnp.float32)
           + jnp.dot(t[Bq:], wf_ref[D:], preferred_element_type=jnp.float32)
           + bf_ref[...])
    out_ref[...] = jnp.maximum(hid, 0.0)


def _head(ga, gnf, W_root, b_gnn, W_fc, b_fc):
    B2 = ga.shape[1]
    H = W_fc.shape[1]
    return pl.pallas_call(
        _head_body,
        out_shape=jax.ShapeDtypeStruct((B2 // 2, H), jnp.float32),
    )(ga, gnf, W_root, b_gnn.reshape(1, -1), W_fc, b_fc.reshape(1, -1))


def kernel(x, node_feat, edge_index, edge_type, nest_tensor, food_tensor,
           W_rel, W_root, b_gnn, W_fc, b_fc):
    N, D = node_feat.shape
    R = W_rel.shape[0]
    E = edge_type.shape[0]

    src = edge_index[0].astype(jnp.int32)
    dst = edge_index[1].astype(jnp.int32)
    et = edge_type.astype(jnp.int32)

    x_rel = _rel_transform(node_feat, W_rel).reshape(R * N, D)

    T16 = -(-E // (NS * C))
    NA = max(2, 2 * round(T16 * 0.56 / 2))
    NB = max(2, 2 * (-(-(T16 - NA) // 2)))
    T = NS * (NA + NB)
    pad = T * C - E
    AGG_ROWS = -(-N // (64 * NS)) * (64 * NS)

    gidx = jnp.concatenate([et * N + src,
                            jnp.zeros((pad,), jnp.int32)]).reshape(T, C)
    didx = jnp.concatenate([dst,
                            jnp.full((pad,), AGG_ROWS - 1, jnp.int32)
                            ]).reshape(T, C)
    qidx = jnp.concatenate([nest_tensor.astype(jnp.int32),
                            food_tensor.astype(jnp.int32)])

    ga, gnf = _sc_aggregate(x_rel, gidx, didx, qidx, node_feat,
                            AGG_ROWS, NA, NB)

    return _head(ga, gnf, W_root, b_gnn, W_fc, b_fc)
